# Initial kernel scaffold; baseline (speedup 1.0000x reference)
import jax
import jax.numpy as jnp
from jax.experimental import pallas as pl


def kernel(user_table, item_table, users, items):
    raise NotImplementedError("write your pallas kernel here")


# speedup vs baseline: 1.0424x; 1.0424x over previous
import functools

import jax
import jax.numpy as jnp
from jax import lax
from jax.experimental import pallas as pl
from jax.experimental.pallas import tpu as pltpu
from jax.experimental.pallas import tpu_sc as plsc

B = 16384
D = 128
NC = 2
NS = 16
NW = NC * NS
BPW = B // NW
CH = 128
K = BPW // CH


def _body(utab, itab, uidx_hbm, iidx_hbm, out, uidx, iidx, urows, irows,
          su, si):
    wid = lax.axis_index("s") * NC + lax.axis_index("c")
    base = wid * BPW
    pltpu.sync_copy(uidx_hbm.at[wid], uidx)
    pltpu.sync_copy(iidx_hbm.at[wid], iidx)
    for j in range(K):
        cu = pltpu.async_copy(utab.at[uidx.at[j]], urows, su)
        ci = pltpu.async_copy(itab.at[iidx.at[j]], irows, si)
        cu.wait()
        pltpu.sync_copy(urows, out.at[pl.ds(base + j * CH, CH), 0])
        ci.wait()
        pltpu.sync_copy(irows, out.at[pl.ds(base + j * CH, CH), 1])


@jax.jit
def _gather_concat(user_table, item_table, users, items):
    mesh = plsc.VectorSubcoreMesh(core_axis_name="c", subcore_axis_name="s")
    f = functools.partial(
        pl.kernel,
        mesh=mesh,
        out_type=jax.ShapeDtypeStruct((B, 2, D), jnp.float32),
        scratch_types=[
            pltpu.VMEM((K, CH), jnp.int32),
            pltpu.VMEM((K, CH), jnp.int32),
            pltpu.VMEM((CH, D), jnp.float32),
            pltpu.VMEM((CH, D), jnp.float32),
            pltpu.SemaphoreType.DMA,
            pltpu.SemaphoreType.DMA,
        ],
    )(_body)
    return f(user_table, item_table,
             users.reshape(NW, K, CH), items.reshape(NW, K, CH))


def kernel(user_table, item_table, users, items):
    out = _gather_concat(user_table, item_table,
                         users.astype(jnp.int32), items.astype(jnp.int32))
    return out.reshape(B, 2 * D)


# speedup vs baseline: 1.1045x; 1.0596x over previous
import functools

import jax
import jax.numpy as jnp
from jax import lax
from jax.experimental import pallas as pl
from jax.experimental.pallas import tpu as pltpu
from jax.experimental.pallas import tpu_sc as plsc

B = 16384
D = 128
NC = 2
NS = 16
NW = NC * NS
BPW = B // NW
CH = 128
K = BPW // CH


NBUF = 3


def _body(utab, itab, uidx_hbm, iidx_hbm, out, uidx, iidx,
          u0, u1, u2, i0, i1, i2, *sems):
    wid = lax.axis_index("s") * NC + lax.axis_index("c")
    base = wid * BPW
    ub, ib = (u0, u1, u2), (i0, i1, i2)
    sgu, sgi, swu, swi = (sems[0:3], sems[3:6], sems[6:9], sems[9:12])
    pltpu.sync_copy(uidx_hbm.at[wid], uidx)
    pltpu.sync_copy(iidx_hbm.at[wid], iidx)

    def gather(j, p):
        return (pltpu.async_copy(utab.at[uidx.at[j]], ub[p], sgu[p]),
                pltpu.async_copy(itab.at[iidx.at[j]], ib[p], sgi[p]))

    gu = [None] * K
    gi = [None] * K
    wu = [None] * K
    wi = [None] * K
    for j in range(min(NBUF, K)):
        gu[j], gi[j] = gather(j, j % NBUF)
    for j in range(K):
        p = j % NBUF
        gu[j].wait()
        wu[j] = pltpu.async_copy(ub[p], out.at[pl.ds(base + j * CH, CH), 0],
                                 swu[p])
        gi[j].wait()
        wi[j] = pltpu.async_copy(ib[p], out.at[pl.ds(base + j * CH, CH), 1],
                                 swi[p])
        nxt = j + NBUF
        if nxt < K:
            wu[j].wait()
            wi[j].wait()
            gu[nxt], gi[nxt] = gather(nxt, p)
    for j in range(max(0, K - NBUF), K):
        if wu[j] is not None:
            wu[j].wait()
            wi[j].wait()


@jax.jit
def _gather_concat(user_table, item_table, users, items):
    mesh = plsc.VectorSubcoreMesh(core_axis_name="c", subcore_axis_name="s")
    f = functools.partial(
        pl.kernel,
        mesh=mesh,
        out_type=jax.ShapeDtypeStruct((B, 2, D), jnp.float32),
        scratch_types=(
            [pltpu.VMEM((K, CH), jnp.int32)] * 2
            + [pltpu.VMEM((CH, D), jnp.float32)] * (2 * NBUF)
            + [pltpu.SemaphoreType.DMA] * (4 * NBUF)
        ),
    )(_body)
    return f(user_table, item_table,
             users.reshape(NW, K, CH), items.reshape(NW, K, CH))


def kernel(user_table, item_table, users, items):
    out = _gather_concat(user_table, item_table,
                         users.astype(jnp.int32), items.astype(jnp.int32))
    return out.reshape(B, 2 * D)


# speedup vs baseline: 1.7810x; 1.6124x over previous
import functools

import jax
import jax.numpy as jnp
from jax import lax
from jax.experimental import pallas as pl
from jax.experimental.pallas import tpu as pltpu
from jax.experimental.pallas import tpu_sc as plsc

B = 16384
D = 128
NC = 2
NS = 16
NW = NC * NS
BPW = B // NW
CH = 128
K = BPW // CH
NBUF = 3


def _body(utab, itab, uidx_hbm, iidx_hbm, out, uidx, iidx,
          u0, u1, u2, i0, i1, i2, *sems):
    wid = lax.axis_index("s") * NC + lax.axis_index("c")
    base = wid * BPW
    ub, ib = (u0, u1, u2), (i0, i1, i2)
    sgu, sgi, swu, swi = (sems[0:3], sems[3:6], sems[6:9], sems[9:12])
    pltpu.sync_copy(uidx_hbm.at[pl.ds(base, BPW)], uidx)
    pltpu.sync_copy(iidx_hbm.at[pl.ds(base, BPW)], iidx)

    def gather(j, p):
        sl = pl.ds(j * CH, CH)
        return (pltpu.async_copy(utab.at[uidx.at[sl]], ub[p], sgu[p]),
                pltpu.async_copy(itab.at[iidx.at[sl]], ib[p], sgi[p]))

    gu = [None] * K
    gi = [None] * K
    wu = [None] * K
    wi = [None] * K
    for j in range(min(NBUF, K)):
        gu[j], gi[j] = gather(j, j % NBUF)
    for j in range(K):
        p = j % NBUF
        rows = pl.ds(base + j * CH, CH)
        gu[j].wait()
        wu[j] = pltpu.async_copy(ub[p], out.at[rows, pl.ds(0, D)], swu[p])
        gi[j].wait()
        wi[j] = pltpu.async_copy(ib[p], out.at[rows, pl.ds(D, D)], swi[p])
        nxt = j + NBUF
        if nxt < K:
            wu[j].wait()
            wi[j].wait()
            gu[nxt], gi[nxt] = gather(nxt, p)
    for j in range(max(0, K - NBUF), K):
        wu[j].wait()
        wi[j].wait()


@jax.jit
def _gather_concat(user_table, item_table, users, items):
    f = functools.partial(
        pl.kernel,
        mesh=plsc.VectorSubcoreMesh(core_axis_name="c", subcore_axis_name="s"),
        out_type=jax.ShapeDtypeStruct((B, 2 * D), jnp.float32),
        scratch_types=(
            [pltpu.VMEM((BPW,), jnp.int32)] * 2
            + [pltpu.VMEM((CH, D), jnp.float32)] * (2 * NBUF)
            + [pltpu.SemaphoreType.DMA] * (4 * NBUF)
        ),
    )(_body)
    return f(user_table, item_table, users, items)


def kernel(user_table, item_table, users, items):
    return _gather_concat(user_table, item_table,
                          users.astype(jnp.int32), items.astype(jnp.int32))
